# baseline (speedup 1.0000x reference)
import functools

import jax
import jax.numpy as jnp
from jax import lax
from jax.experimental import pallas as pl
from jax.experimental.pallas import tpu as pltpu
from jax.experimental.pallas import tpu_sc as plsc

D = 128
G = 128
NC, NS = 2, 16
NW = NC * NS
NT = 3


def _sc_fused(idx2d, lab2d, table, pos2d, seg_table, L):
    n_groups = idx2d.shape[0]
    g_per_w = n_groups // NW
    gpl = L // G
    lpt = L // NS
    S = seg_table.shape[0]
    mesh = plsc.VectorSubcoreMesh(core_axis_name="c", subcore_axis_name="s")

    @functools.partial(
        pl.kernel,
        mesh=mesh,
        out_type=jax.ShapeDtypeStruct((n_groups * G, D), jnp.float32),
        scratch_types=[
            pltpu.VMEM((g_per_w, G), jnp.int32),
            pltpu.VMEM((g_per_w, G), jnp.int32),
            pltpu.VMEM((S, D), jnp.float32),
            pltpu.VMEM_SHARED((S * L, D), jnp.float32),
            pltpu.VMEM((G,), jnp.int32),
            pltpu.VMEM((G, D), jnp.float32),
        ]
        + [pltpu.VMEM((G, D), jnp.float32) for _ in range(NT)]
        + [pltpu.SemaphoreType.DMA for _ in range(NT)]
        + [pltpu.SemaphoreType.DMA]
        + [pltpu.SemaphoreType.DMA for _ in range(NT)],
    )
    def k(idx_hbm, lab_hbm, table_hbm, pos_hbm, seg_hbm, out_hbm,
          idx_v, lab_v, seg_v, base_sh, cidx, bas, *rest):
        tok = rest[:NT]
        sgt = rest[NT:2 * NT]
        sgb = rest[2 * NT]
        so = rest[2 * NT + 1:]

        sid = lax.axis_index("s")
        wid = sid * NC + lax.axis_index("c")
        wbase = wid * g_per_w

        pltpu.sync_copy(idx_hbm.at[pl.ds(wbase, g_per_w)], idx_v)
        pltpu.sync_copy(lab_hbm.at[pl.ds(wbase, g_per_w)], lab_v)

        for b in range(NT - 1):
            pltpu.async_copy(table_hbm.at[idx_v.at[b]], tok[b], sgt[b])

        pltpu.sync_copy(seg_hbm, seg_v)
        for c in range(lpt // 16):
            cidx[pl.ds(c * 16, 16)] = (
                sid * lpt + c * 16 + lax.iota(jnp.int32, 16))
        pltpu.async_copy(pos_hbm.at[cidx], bas, sgb).wait()
        for s in range(S):
            if s == 0:
                d16 = [seg_v[0, pl.ds(c * 16, 16)] for c in range(D // 16)]
            else:
                d16 = [seg_v[s, pl.ds(c * 16, 16)]
                       - seg_v[s - 1, pl.ds(c * 16, 16)]
                       for c in range(D // 16)]

            def brow(r, c2, d16=d16):
                for c in range(D // 16):
                    sl = pl.ds(c * 16, 16)
                    bas[r, sl] = bas[r, sl] + d16[c]
                return c2

            lax.fori_loop(0, lpt, brow, 0)
            pltpu.sync_copy(bas, base_sh.at[pl.ds(s * L + sid * lpt, lpt)])
        plsc.subcore_barrier()

        def comp_cidx(g):
            l0 = lax.rem(wbase + g, gpl) * G
            for c in range(G // 16):
                lab16 = lab_v[g, pl.ds(c * 16, 16)]
                cidx[pl.ds(c * 16, 16)] = (
                    lab16 * L + (l0 + c * 16) + lax.iota(jnp.int32, 16))

        comp_cidx(0)
        pltpu.async_copy(base_sh.at[cidx], bas, sgb)

        for g in range(g_per_w):
            bt = g % NT
            pltpu.make_async_copy(table_hbm.at[idx_v.at[0]], tok[bt],
                                  sgt[bt]).wait()
            pltpu.make_async_copy(base_sh.at[cidx], bas, sgb).wait()

            def row(r, c2, bt=bt):
                for u in range(2):
                    for c in range(D // 16):
                        sl = pl.ds(c * 16, 16)
                        plsc.addupdate(tok[bt].at[2 * r + u, sl],
                                       bas[2 * r + u, sl])
                return c2

            lax.fori_loop(0, G // 2, row, 0)
            if g + 1 < g_per_w:
                comp_cidx(g + 1)
                pltpu.async_copy(base_sh.at[cidx], bas, sgb)
            pltpu.async_copy(tok[bt], out_hbm.at[pl.ds((wbase + g) * G, G)],
                             so[bt])
            if g + NT - 1 < g_per_w:
                bp = (g + NT - 1) % NT
                if g + NT - 1 >= NT:
                    pltpu.make_async_copy(tok[bp], out_hbm.at[pl.ds(0, G)],
                                          so[bp]).wait()
                pltpu.async_copy(table_hbm.at[idx_v.at[g + NT - 1]], tok[bp],
                                 sgt[bp])

        for b in range(NT):
            pltpu.make_async_copy(tok[b], out_hbm.at[pl.ds(0, G)],
                                  so[b]).wait()

    return k(idx2d, lab2d, table, pos2d, seg_table)


def kernel(sequence, segment_label, token_table, segment_table, pos_emb):
    B, L = sequence.shape
    N = B * L
    idx2d = sequence.reshape(N // G, G)
    lab2d = segment_label.reshape(N // G, G)
    pos2d = pos_emb.reshape(L, D)
    out = _sc_fused(idx2d, lab2d, token_table, pos2d, segment_table, L)
    return out.reshape(B, L, D)
